# baseline (speedup 1.0000x reference)
import functools

import jax
import jax.numpy as jnp
from jax import lax
from jax.experimental import pallas as pl
from jax.experimental.pallas import tpu as pltpu
from jax.experimental.pallas import tpu_sc as plsc

N = 16384
NCAT = 26
NNUM = 13
NCOL = NCAT + NNUM
VOCAB = 1000
C = 128
NW = 32
RPW = N // NW
HR = RPW // 2
NCH = 2 * NCAT
IPD = 128
GPC = HR // IPD
NSLOT = 3
LANES = 16
NBLK = 1024

_mesh = plsc.VectorSubcoreMesh(core_axis_name="c", subcore_axis_name="s")


@functools.partial(
    pl.kernel,
    mesh=_mesh,
    out_type=jax.ShapeDtypeStruct((N * NCOL, C), jnp.float32),
    compiler_params=pltpu.CompilerParams(needs_layout_passes=False),
    scratch_types=[
        pltpu.VMEM((RPW * NCAT,), jnp.int32),
        pltpu.VMEM((NCAT * 4, IPD), jnp.int32),
        pltpu.VMEM((NSLOT, HR, C), jnp.float32),
        pltpu.SemaphoreType.DMA((NSLOT,)),
        pltpu.SemaphoreType.DMA((NSLOT,)),
    ],
)
def _encode_cat(fc_hbm, table_hbm, out_hbm, fc_v, idx_v, gbuf, gsem, wsem):
    wid = lax.axis_index("s") * 2 + lax.axis_index("c")
    pltpu.sync_copy(fc_hbm.at[wid], fc_v)
    wbase = wid * RPW

    lane26 = lax.iota(jnp.int32, LANES) * NCAT

    def build_col(col, carry):
        coff = col * VOCAB
        for p in range(4):
            for g in range(IPD // LANES):
                f0 = (p * IPD + g * LANES) * NCAT + col
                v = plsc.load_gather(fc_v, [lane26 + f0])
                idx_v[col * 4 + p, pl.ds(g * LANES, LANES)] = v + coff
        return carry

    lax.fori_loop(0, NCAT, build_col, 0)

    def cat_dst(t):
        return (t // 2) * N + wbase + (t % 2) * HR

    def fire_gathers(t, s):
        for q in range(GPC):
            pltpu.async_copy(table_hbm.at[idx_v.at[t * GPC + q]],
                             gbuf.at[s, pl.ds(q * IPD, IPD)], gsem.at[s])

    fire_gathers(0, 0)
    fire_gathers(1, 1)

    def chunk(t, carry):
        s = t % NSLOT

        @pl.when(t >= 1)
        def _():
            sp = (t + 2) % NSLOT
            pltpu.make_async_copy(
                gbuf.at[sp], out_hbm.at[pl.ds(cat_dst(t - 1), HR)],
                wsem.at[sp]).wait()

        @pl.when(t < NCH - 2)
        def _():
            fire_gathers(t + 2, (t + 2) % NSLOT)

        for q in range(GPC):
            pltpu.make_async_copy(table_hbm.at[idx_v.at[t * GPC + q]],
                                  gbuf.at[s, pl.ds(q * IPD, IPD)],
                                  gsem.at[s]).wait()

        pltpu.async_copy(gbuf.at[s], out_hbm.at[pl.ds(cat_dst(t), HR)],
                         wsem.at[s])
        return carry

    lax.fori_loop(0, NCH, chunk, 0)
    sl = (NCH - 1) % NSLOT
    pltpu.make_async_copy(gbuf.at[sl], out_hbm.at[pl.ds(cat_dst(NCH - 1), HR)],
                          wsem.at[sl]).wait()


def _num_body(buf_ref, fn_ref, w_ref, b_ref, out_ref):
    del buf_ref
    j = pl.program_id(0) // (N // NBLK)
    sel = (lax.broadcasted_iota(jnp.int32, (1, NNUM), 1) == j).astype(jnp.float32)
    fn = jnp.sum(fn_ref[0] * sel, axis=1)
    out_ref[...] = fn[:, None] * w_ref[...][0] + b_ref[...][0]


_num_fill = pl.pallas_call(
    _num_body,
    grid=(NNUM * (N // NBLK),),
    in_specs=[
        pl.BlockSpec(memory_space=pl.ANY),
        pl.BlockSpec((1, NBLK, NNUM), lambda i: (i % (N // NBLK), 0, 0)),
        pl.BlockSpec((1, 1, C), lambda i: (i // (N // NBLK), 0, 0)),
        pl.BlockSpec((1, 1, C), lambda i: (i // (N // NBLK), 0, 0)),
    ],
    out_specs=pl.BlockSpec((NBLK, C), lambda i: (NCAT * (N // NBLK) + i, 0)),
    out_shape=jax.ShapeDtypeStruct((N * NCOL, C), jnp.float32),
    input_output_aliases={0: 0},
)


def kernel(feat_cat, feat_num, emb_tables, lin_weight, lin_bias, num_mean, num_std):
    table = emb_tables.reshape(NCAT * VOCAB, C)
    fc = feat_cat.astype(jnp.int32).reshape(NW, RPW * NCAT)
    inv = 1.0 / num_std
    w_eff = lin_weight * inv[:, None]
    b_eff = lin_bias - (num_mean * inv)[:, None] * lin_weight
    out = _encode_cat(fc, table)
    out = _num_fill(out, feat_num.reshape(N // NBLK, NBLK, NNUM),
                    w_eff[:, None, :], b_eff[:, None, :])
    return out.reshape(NCOL, N, C).transpose(1, 0, 2)


# speedup vs baseline: 1.1732x; 1.1732x over previous
import functools

import jax
import jax.numpy as jnp
from jax import lax
from jax.experimental import pallas as pl
from jax.experimental.pallas import tpu as pltpu
from jax.experimental.pallas import tpu_sc as plsc

N = 16384
NCAT = 26
NNUM = 13
NCOL = NCAT + NNUM
VOCAB = 1000
C = 128
NW = 32
RPW = N // NW
HR = RPW // 2
NCH = 2 * NCAT
IPD = 128
GPC = HR // IPD
NSLOT = 3
LANES = 16
NBLK = 1024

_mesh = plsc.VectorSubcoreMesh(core_axis_name="c", subcore_axis_name="s")


@functools.partial(
    pl.kernel,
    mesh=_mesh,
    out_type=jax.ShapeDtypeStruct((N * NCOL, C), jnp.float32),
    scratch_types=[
        pltpu.VMEM((NCAT * 4, IPD), jnp.int32),
        pltpu.VMEM((NSLOT, HR, C), jnp.float32),
        pltpu.SemaphoreType.DMA((NSLOT,)),
        pltpu.SemaphoreType.DMA((NSLOT,)),
    ],
)
def _encode_cat(table_hbm, idx_hbm, out_hbm, idx_v, gbuf, gsem, wsem):
    wid = lax.axis_index("s") * 2 + lax.axis_index("c")
    pltpu.sync_copy(idx_hbm.at[wid], idx_v)
    wbase = wid * RPW

    def cat_dst(t):
        return (t // 2) * N + wbase + (t % 2) * HR

    def fire_gathers(t, s):
        for q in range(GPC):
            pltpu.async_copy(table_hbm.at[idx_v.at[t * GPC + q]],
                             gbuf.at[s, pl.ds(q * IPD, IPD)], gsem.at[s])

    fire_gathers(0, 0)
    fire_gathers(1, 1)

    def chunk(t, carry):
        s = t % NSLOT

        @pl.when(t >= 1)
        def _():
            sp = (t + 2) % NSLOT
            pltpu.make_async_copy(
                gbuf.at[sp], out_hbm.at[pl.ds(cat_dst(t - 1), HR)],
                wsem.at[sp]).wait()

        @pl.when(t < NCH - 2)
        def _():
            fire_gathers(t + 2, (t + 2) % NSLOT)

        for q in range(GPC):
            pltpu.make_async_copy(table_hbm.at[idx_v.at[t * GPC + q]],
                                  gbuf.at[s, pl.ds(q * IPD, IPD)],
                                  gsem.at[s]).wait()

        pltpu.async_copy(gbuf.at[s], out_hbm.at[pl.ds(cat_dst(t), HR)],
                         wsem.at[s])
        return carry

    lax.fori_loop(0, NCH, chunk, 0)
    sl = (NCH - 1) % NSLOT
    pltpu.make_async_copy(gbuf.at[sl], out_hbm.at[pl.ds(cat_dst(NCH - 1), HR)],
                          wsem.at[sl]).wait()


def _num_body(buf_ref, fn_ref, w_ref, b_ref, out_ref):
    del buf_ref
    fn = fn_ref[...]
    out_ref[...] = fn[0, 0, :, None] * w_ref[...][0] + b_ref[...][0]


_num_fill = pl.pallas_call(
    _num_body,
    grid=(NNUM * (N // NBLK),),
    in_specs=[
        pl.BlockSpec(memory_space=pl.ANY),
        pl.BlockSpec((1, 1, NBLK),
                     lambda i: (i // (N // NBLK) * (N // NBLK)
                                + i % (N // NBLK), 0, 0)),
        pl.BlockSpec((1, 1, C), lambda i: (i // (N // NBLK), 0, 0)),
        pl.BlockSpec((1, 1, C), lambda i: (i // (N // NBLK), 0, 0)),
    ],
    out_specs=pl.BlockSpec((NBLK, C), lambda i: (NCAT * (N // NBLK) + i, 0)),
    out_shape=jax.ShapeDtypeStruct((N * NCOL, C), jnp.float32),
    input_output_aliases={0: 0},
)


def kernel(feat_cat, feat_num, emb_tables, lin_weight, lin_bias, num_mean, num_std):
    table = emb_tables.reshape(NCAT * VOCAB, C)
    offs = jnp.arange(NCAT, dtype=jnp.int32) * VOCAB
    idx = (feat_cat.astype(jnp.int32) + offs[None, :]).T
    idx = idx.reshape(NCAT, NW, 4, IPD).transpose(1, 0, 2, 3)
    idx = idx.reshape(NW, NCAT * 4, IPD)
    inv = 1.0 / num_std
    w_eff = lin_weight * inv[:, None]
    b_eff = lin_bias - (num_mean * inv)[:, None] * lin_weight
    out = _encode_cat(table, idx)
    fnum_blk = feat_num.T.reshape(NNUM * (N // NBLK), 1, NBLK)
    out = _num_fill(out, fnum_blk, w_eff[:, None, :], b_eff[:, None, :])
    return out.reshape(NCOL, N, C).transpose(1, 0, 2)


# speedup vs baseline: 1.5098x; 1.2869x over previous
import functools

import jax
import jax.numpy as jnp
from jax import lax
from jax.experimental import pallas as pl
from jax.experimental.pallas import tpu as pltpu
from jax.experimental.pallas import tpu_sc as plsc

N = 16384
NCAT = 26
NNUM = 13
NCOL = NCAT + NNUM
VOCAB = 1000
C = 128
NW = 32
RPW = N // NW
HR = RPW // 2
NCH = 2 * NCAT
IPD = 128
GPC = HR // IPD
NSLOT = 3
LANES = 16
NBLK = 4096

_mesh = plsc.VectorSubcoreMesh(core_axis_name="c", subcore_axis_name="s")


@functools.partial(
    pl.kernel,
    mesh=_mesh,
    out_type=jax.ShapeDtypeStruct((N * NCOL, C), jnp.float32),
    scratch_types=[
        pltpu.VMEM((NCAT * 4, IPD), jnp.int32),
        pltpu.VMEM((NSLOT, HR, C), jnp.float32),
        pltpu.SemaphoreType.DMA((NSLOT,)),
        pltpu.SemaphoreType.DMA((NSLOT,)),
    ],
)
def _encode_cat(table_hbm, idx_hbm, out_hbm, idx_v, gbuf, gsem, wsem):
    wid = lax.axis_index("s") * 2 + lax.axis_index("c")
    pltpu.sync_copy(idx_hbm.at[wid], idx_v)
    wbase = wid * RPW

    def cat_dst(t):
        return (t // 2) * N + wbase + (t % 2) * HR

    def fire_gathers(t, s):
        for q in range(GPC):
            pltpu.async_copy(table_hbm.at[idx_v.at[t * GPC + q]],
                             gbuf.at[s, pl.ds(q * IPD, IPD)], gsem.at[s])

    fire_gathers(0, 0)
    fire_gathers(1, 1)

    def chunk(t, carry):
        s = t % NSLOT

        @pl.when(t >= 1)
        def _():
            sp = (t + 2) % NSLOT
            pltpu.make_async_copy(
                gbuf.at[sp], out_hbm.at[pl.ds(cat_dst(t - 1), HR)],
                wsem.at[sp]).wait()

        @pl.when(t < NCH - 2)
        def _():
            fire_gathers(t + 2, (t + 2) % NSLOT)

        for q in range(GPC):
            pltpu.make_async_copy(table_hbm.at[idx_v.at[t * GPC + q]],
                                  gbuf.at[s, pl.ds(q * IPD, IPD)],
                                  gsem.at[s]).wait()

        pltpu.async_copy(gbuf.at[s], out_hbm.at[pl.ds(cat_dst(t), HR)],
                         wsem.at[s])
        return carry

    lax.fori_loop(0, NCH, chunk, 0)
    sl = (NCH - 1) % NSLOT
    pltpu.make_async_copy(gbuf.at[sl], out_hbm.at[pl.ds(cat_dst(NCH - 1), HR)],
                          wsem.at[sl]).wait()


def _num_body(buf_ref, fn_ref, w_ref, b_ref, out_ref):
    del buf_ref
    fn = fn_ref[...]
    out_ref[...] = fn[0, 0, :, None] * w_ref[...][0] + b_ref[...][0]


_num_fill = pl.pallas_call(
    _num_body,
    grid=(NNUM * (N // NBLK),),
    in_specs=[
        pl.BlockSpec(memory_space=pl.ANY),
        pl.BlockSpec((1, 1, NBLK),
                     lambda i: (i // (N // NBLK) * (N // NBLK)
                                + i % (N // NBLK), 0, 0)),
        pl.BlockSpec((1, 1, C), lambda i: (i // (N // NBLK), 0, 0)),
        pl.BlockSpec((1, 1, C), lambda i: (i // (N // NBLK), 0, 0)),
    ],
    out_specs=pl.BlockSpec((NBLK, C), lambda i: (NCAT * (N // NBLK) + i, 0)),
    out_shape=jax.ShapeDtypeStruct((N * NCOL, C), jnp.float32),
    input_output_aliases={0: 0},
)


def kernel(feat_cat, feat_num, emb_tables, lin_weight, lin_bias, num_mean, num_std):
    table = emb_tables.reshape(NCAT * VOCAB, C)
    offs = jnp.arange(NCAT, dtype=jnp.int32) * VOCAB
    idx = (feat_cat.astype(jnp.int32) + offs[None, :]).T
    idx = idx.reshape(NCAT, NW, 4, IPD).transpose(1, 0, 2, 3)
    idx = idx.reshape(NW, NCAT * 4, IPD)
    inv = 1.0 / num_std
    w_eff = lin_weight * inv[:, None]
    b_eff = lin_bias - (num_mean * inv)[:, None] * lin_weight
    out = _encode_cat(table, idx)
    fnum_blk = feat_num.T.reshape(NNUM * (N // NBLK), 1, NBLK)
    out = _num_fill(out, fnum_blk, w_eff[:, None, :], b_eff[:, None, :])
    return out.reshape(NCOL, N, C).transpose(1, 0, 2)


# speedup vs baseline: 1.5863x; 1.0507x over previous
import functools

import jax
import jax.numpy as jnp
from jax import lax
from jax.experimental import pallas as pl
from jax.experimental.pallas import tpu as pltpu
from jax.experimental.pallas import tpu_sc as plsc

N = 16384
NCAT = 26
NNUM = 13
NCOL = NCAT + NNUM
VOCAB = 1000
C = 128
NW = 32
RPW = N // NW
HR = RPW // 2
NCH = 2 * NCAT
IPD = 128
GPC = HR // IPD
NSLOT = 3
LANES = 16
NBLK = 8192

_mesh = plsc.VectorSubcoreMesh(core_axis_name="c", subcore_axis_name="s")


@functools.partial(
    pl.kernel,
    mesh=_mesh,
    out_type=jax.ShapeDtypeStruct((N * NCOL, C), jnp.float32),
    scratch_types=[
        pltpu.VMEM((NCAT * 4, IPD), jnp.int32),
        pltpu.VMEM((NSLOT, HR, C), jnp.float32),
        pltpu.SemaphoreType.DMA((NSLOT,)),
        pltpu.SemaphoreType.DMA((NSLOT,)),
    ],
)
def _encode_cat(table_hbm, idx_hbm, out_hbm, idx_v, gbuf, gsem, wsem):
    wid = lax.axis_index("s") * 2 + lax.axis_index("c")
    pltpu.sync_copy(idx_hbm.at[wid], idx_v)
    wbase = wid * RPW

    def cat_dst(t):
        return (t // 2) * N + wbase + (t % 2) * HR

    def fire_gathers(t, s):
        for q in range(GPC):
            pltpu.async_copy(table_hbm.at[idx_v.at[t * GPC + q]],
                             gbuf.at[s, pl.ds(q * IPD, IPD)], gsem.at[s])

    fire_gathers(0, 0)
    fire_gathers(1, 1)

    def chunk(t, carry):
        s = t % NSLOT

        @pl.when(t >= 1)
        def _():
            sp = (t + 2) % NSLOT
            pltpu.make_async_copy(
                gbuf.at[sp], out_hbm.at[pl.ds(cat_dst(t - 1), HR)],
                wsem.at[sp]).wait()

        @pl.when(t < NCH - 2)
        def _():
            fire_gathers(t + 2, (t + 2) % NSLOT)

        for q in range(GPC):
            pltpu.make_async_copy(table_hbm.at[idx_v.at[t * GPC + q]],
                                  gbuf.at[s, pl.ds(q * IPD, IPD)],
                                  gsem.at[s]).wait()

        pltpu.async_copy(gbuf.at[s], out_hbm.at[pl.ds(cat_dst(t), HR)],
                         wsem.at[s])
        return carry

    lax.fori_loop(0, NCH, chunk, 0)
    sl = (NCH - 1) % NSLOT
    pltpu.make_async_copy(gbuf.at[sl], out_hbm.at[pl.ds(cat_dst(NCH - 1), HR)],
                          wsem.at[sl]).wait()


def _num_body(buf_ref, fn_ref, w_ref, b_ref, out_ref):
    del buf_ref
    fn = fn_ref[...]
    out_ref[...] = fn[0, 0, :, None] * w_ref[...][0] + b_ref[...][0]


_num_fill = pl.pallas_call(
    _num_body,
    grid=(NNUM * (N // NBLK),),
    in_specs=[
        pl.BlockSpec(memory_space=pl.ANY),
        pl.BlockSpec((1, 1, NBLK),
                     lambda i: (i // (N // NBLK) * (N // NBLK)
                                + i % (N // NBLK), 0, 0)),
        pl.BlockSpec((1, 1, C), lambda i: (i // (N // NBLK), 0, 0)),
        pl.BlockSpec((1, 1, C), lambda i: (i // (N // NBLK), 0, 0)),
    ],
    out_specs=pl.BlockSpec((NBLK, C), lambda i: (NCAT * (N // NBLK) + i, 0)),
    out_shape=jax.ShapeDtypeStruct((N * NCOL, C), jnp.float32),
    input_output_aliases={0: 0},
)


def kernel(feat_cat, feat_num, emb_tables, lin_weight, lin_bias, num_mean, num_std):
    table = emb_tables.reshape(NCAT * VOCAB, C)
    offs = jnp.arange(NCAT, dtype=jnp.int32) * VOCAB
    idx = (feat_cat.astype(jnp.int32) + offs[None, :]).T
    idx = idx.reshape(NCAT, NW, 4, IPD).transpose(1, 0, 2, 3)
    idx = idx.reshape(NW, NCAT * 4, IPD)
    inv = 1.0 / num_std
    w_eff = lin_weight * inv[:, None]
    b_eff = lin_bias - (num_mean * inv)[:, None] * lin_weight
    out = _encode_cat(table, idx)
    fnum_blk = feat_num.T.reshape(NNUM * (N // NBLK), 1, NBLK)
    out = _num_fill(out, fnum_blk, w_eff[:, None, :], b_eff[:, None, :])
    return out.reshape(NCOL, N, C).transpose(1, 0, 2)


# speedup vs baseline: 1.6075x; 1.0134x over previous
import functools

import jax
import jax.numpy as jnp
from jax import lax
from jax.experimental import pallas as pl
from jax.experimental.pallas import tpu as pltpu
from jax.experimental.pallas import tpu_sc as plsc

N = 16384
NCAT = 26
NNUM = 13
NCOL = NCAT + NNUM
VOCAB = 1000
C = 128
NW = 32
RPW = N // NW
HR = RPW // 2
NCH = 2 * NCAT
IPD = 128
GPC = HR // IPD
NSLOT = 3
LANES = 16
NBLK = 16384

_mesh = plsc.VectorSubcoreMesh(core_axis_name="c", subcore_axis_name="s")


@functools.partial(
    pl.kernel,
    mesh=_mesh,
    out_type=jax.ShapeDtypeStruct((N * NCOL, C), jnp.float32),
    scratch_types=[
        pltpu.VMEM((NCAT * 4, IPD), jnp.int32),
        pltpu.VMEM((NSLOT, HR, C), jnp.float32),
        pltpu.SemaphoreType.DMA((NSLOT,)),
        pltpu.SemaphoreType.DMA((NSLOT,)),
    ],
)
def _encode_cat(table_hbm, idx_hbm, out_hbm, idx_v, gbuf, gsem, wsem):
    wid = lax.axis_index("s") * 2 + lax.axis_index("c")
    pltpu.sync_copy(idx_hbm.at[wid], idx_v)
    wbase = wid * RPW

    def cat_dst(t):
        return (t // 2) * N + wbase + (t % 2) * HR

    def fire_gathers(t, s):
        for q in range(GPC):
            pltpu.async_copy(table_hbm.at[idx_v.at[t * GPC + q]],
                             gbuf.at[s, pl.ds(q * IPD, IPD)], gsem.at[s])

    fire_gathers(0, 0)
    fire_gathers(1, 1)

    def chunk(t, carry):
        s = t % NSLOT

        @pl.when(t >= 1)
        def _():
            sp = (t + 2) % NSLOT
            pltpu.make_async_copy(
                gbuf.at[sp], out_hbm.at[pl.ds(cat_dst(t - 1), HR)],
                wsem.at[sp]).wait()

        @pl.when(t < NCH - 2)
        def _():
            fire_gathers(t + 2, (t + 2) % NSLOT)

        for q in range(GPC):
            pltpu.make_async_copy(table_hbm.at[idx_v.at[t * GPC + q]],
                                  gbuf.at[s, pl.ds(q * IPD, IPD)],
                                  gsem.at[s]).wait()

        pltpu.async_copy(gbuf.at[s], out_hbm.at[pl.ds(cat_dst(t), HR)],
                         wsem.at[s])
        return carry

    lax.fori_loop(0, NCH, chunk, 0)
    sl = (NCH - 1) % NSLOT
    pltpu.make_async_copy(gbuf.at[sl], out_hbm.at[pl.ds(cat_dst(NCH - 1), HR)],
                          wsem.at[sl]).wait()


def _num_body(buf_ref, fn_ref, w_ref, b_ref, out_ref):
    del buf_ref
    fn = fn_ref[...]
    out_ref[...] = fn[0, 0, :, None] * w_ref[...][0] + b_ref[...][0]


_num_fill = pl.pallas_call(
    _num_body,
    grid=(NNUM * (N // NBLK),),
    in_specs=[
        pl.BlockSpec(memory_space=pl.ANY),
        pl.BlockSpec((1, 1, NBLK),
                     lambda i: (i // (N // NBLK) * (N // NBLK)
                                + i % (N // NBLK), 0, 0)),
        pl.BlockSpec((1, 1, C), lambda i: (i // (N // NBLK), 0, 0)),
        pl.BlockSpec((1, 1, C), lambda i: (i // (N // NBLK), 0, 0)),
    ],
    out_specs=pl.BlockSpec((NBLK, C), lambda i: (NCAT * (N // NBLK) + i, 0)),
    out_shape=jax.ShapeDtypeStruct((N * NCOL, C), jnp.float32),
    input_output_aliases={0: 0},
)


def kernel(feat_cat, feat_num, emb_tables, lin_weight, lin_bias, num_mean, num_std):
    table = emb_tables.reshape(NCAT * VOCAB, C)
    offs = jnp.arange(NCAT, dtype=jnp.int32) * VOCAB
    idx = (feat_cat.astype(jnp.int32) + offs[None, :]).T
    idx = idx.reshape(NCAT, NW, 4, IPD).transpose(1, 0, 2, 3)
    idx = idx.reshape(NW, NCAT * 4, IPD)
    inv = 1.0 / num_std
    w_eff = lin_weight * inv[:, None]
    b_eff = lin_bias - (num_mean * inv)[:, None] * lin_weight
    out = _encode_cat(table, idx)
    fnum_blk = feat_num.T.reshape(NNUM * (N // NBLK), 1, NBLK)
    out = _num_fill(out, fnum_blk, w_eff[:, None, :], b_eff[:, None, :])
    return out.reshape(NCOL, N, C).transpose(1, 0, 2)
